# baseline (speedup 1.0000x reference)
import functools
import jax
import jax.numpy as jnp
from jax import lax
from jax.experimental import pallas as pl
from jax.experimental.pallas import tpu as pltpu
from jax.experimental.pallas import tpu_sc as plsc

BN_BLK = 512
S = 1024
SC_CHUNK = 2048
HI = 3.4e38


def _top3(dists, iota_s):
    d = dists
    vals, idxs = [], []
    for k in range(3):
        m = jnp.min(d, axis=0)
        is_min = d == m[None, :]
        ik = jnp.min(jnp.where(is_min, iota_s, S), axis=0)
        vals.append(m)
        idxs.append(ik)
        if k < 2:
            d = jnp.where(iota_s == ik[None, :], HI, d)
    return vals, idxs


def _p1_kernel(x1_ref, x2_ref, idx_ref, w_ref):
    x1 = x1_ref[0]
    x2 = x2_ref[0]
    dots = jax.lax.dot_general(x2, x1, (((0,), (0,)), ((), ())),
                               preferred_element_type=jnp.float32,
                               precision=lax.Precision.DEFAULT)
    x1sq = (x1[0] * x1[0] + x1[1] * x1[1]) + x1[2] * x1[2]
    x2sq = (x2[0] * x2[0] + x2[1] * x2[1]) + x2[2] * x2[2]
    dists = (-2.0 * dots + x1sq[None, :]) + x2sq[:, None]

    iota_s = lax.broadcasted_iota(jnp.int32, dists.shape, 0)
    vals, idxs = _top3(dists, iota_s)

    r0 = 1.0 / (vals[0] + 1e-8)
    r1 = 1.0 / (vals[1] + 1e-8)
    r2 = 1.0 / (vals[2] + 1e-8)
    norm = r0 + r1 + r2

    idx_ref[0, 0] = idxs[0]
    idx_ref[0, 1] = idxs[1]
    idx_ref[0, 2] = idxs[2]
    w_ref[0, 0] = r0 / norm
    w_ref[0, 1] = r1 / norm
    w_ref[0, 2] = r2 / norm


def _sc_interp(p2, idxT, wT, B, D2, N):
    mesh = plsc.VectorSubcoreMesh(core_axis_name="c", subcore_axis_name="s")

    @functools.partial(
        pl.kernel,
        out_type=jax.ShapeDtypeStruct((B, D2, N), jnp.float32),
        mesh=mesh,
        compiler_params=pltpu.CompilerParams(use_tc_tiling_on_sc=False,
                                             needs_layout_passes=False),
        scratch_types=[
            pltpu.VMEM((16, S), jnp.float32),
            pltpu.VMEM((3, SC_CHUNK), jnp.int32),
            pltpu.VMEM((3, SC_CHUNK), jnp.float32),
            pltpu.VMEM((16, SC_CHUNK), jnp.float32),
        ],
    )
    def k(p2_hbm, idx_hbm, w_hbm, out_hbm, tbl_v, idx_v, w_v, o_v):
        wid = lax.axis_index("s") * 2 + lax.axis_index("c")
        b = wid % 2
        c = wid // 2
        pltpu.sync_copy(p2_hbm.at[b, pl.ds(c * 16, 16), :], tbl_v)
        fulls = [jax.lax.broadcast(jnp.int32(ch), (16,)) for ch in range(16)]

        def do_group(g, _):
            sl = pl.ds(g * 16, 16)
            i0 = idx_v[0, sl]
            i1 = idx_v[1, sl]
            i2 = idx_v[2, sl]
            w0 = w_v[0, sl]
            w1 = w_v[1, sl]
            w2 = w_v[2, sl]
            for ch in range(16):
                g0 = plsc.load_gather(tbl_v, [fulls[ch], i0])
                g1 = plsc.load_gather(tbl_v, [fulls[ch], i1])
                g2 = plsc.load_gather(tbl_v, [fulls[ch], i2])
                o_v[ch, sl] = (g0 * w0 + g1 * w1) + g2 * w2
            return ()

        for t in range(N // SC_CHUNK):
            tsl = pl.ds(t * SC_CHUNK, SC_CHUNK)
            pltpu.sync_copy(idx_hbm.at[b, :, tsl], idx_v)
            pltpu.sync_copy(w_hbm.at[b, :, tsl], w_v)
            lax.fori_loop(0, SC_CHUNK // 16, do_group, (), unroll=1)
            pltpu.sync_copy(o_v, out_hbm.at[b, pl.ds(c * 16, 16), tsl])

    return k(p2, idxT, wT)


def _p2_kernel(p1_ref, it_ref, w1_ref, b1_ref, h1_ref, st1_ref, acc_ref, *, nj):
    b = pl.program_id(0)
    j = pl.program_id(1)
    W1 = w1_ref[...]
    h1 = (jax.lax.dot_general(W1[:, :128], p1_ref[0], (((1,), (0,)), ((), ())),
                              preferred_element_type=jnp.float32,
                              precision=lax.Precision.DEFAULT)
          + jax.lax.dot_general(W1[:, 128:], it_ref[0], (((1,), (0,)), ((), ())),
                                preferred_element_type=jnp.float32,
                                precision=lax.Precision.DEFAULT)
          + b1_ref[...])
    h1_ref[0] = h1

    @pl.when(jnp.logical_and(b == 0, j == 0))
    def _init():
        acc_ref[...] = jnp.zeros_like(acc_ref)

    acc_ref[0] += h1
    acc_ref[1] += h1 * h1

    @pl.when(jnp.logical_and(b == pl.num_programs(0) - 1, j == nj - 1))
    def _fin():
        st1_ref[0, :] = jnp.sum(acc_ref[0], axis=1)
        st1_ref[1, :] = jnp.sum(acc_ref[1], axis=1)


def _p3_kernel(h1_ref, w2_ref, b2_ref, sc1_ref, sh1_ref, h2_ref, st2_ref,
               acc_ref, *, nj):
    b = pl.program_id(0)
    j = pl.program_id(1)
    a1 = jnp.maximum(sc1_ref[...] * h1_ref[0] + sh1_ref[...], 0.0)
    h2 = (jax.lax.dot_general(w2_ref[...], a1, (((1,), (0,)), ((), ())),
                              preferred_element_type=jnp.float32,
                              precision=lax.Precision.DEFAULT)
          + b2_ref[...])
    h2_ref[0] = h2

    @pl.when(jnp.logical_and(b == 0, j == 0))
    def _init():
        acc_ref[...] = jnp.zeros_like(acc_ref)

    acc_ref[0] += h2
    acc_ref[1] += h2 * h2

    @pl.when(jnp.logical_and(b == pl.num_programs(0) - 1, j == nj - 1))
    def _fin():
        st2_ref[0, :] = jnp.sum(acc_ref[0], axis=1)
        st2_ref[1, :] = jnp.sum(acc_ref[1], axis=1)


def _p4_kernel(h2_ref, sc2_ref, sh2_ref, o_ref):
    o_ref[0] = jnp.maximum(sc2_ref[...] * h2_ref[0] + sh2_ref[...], 0.0)


def _affine(stats, g, be, count):
    mean = stats[0] / count
    var = stats[1] / count - mean * mean
    scale = g / jnp.sqrt(var + 1e-5)
    shift = be - scale * mean
    return scale[:, None], shift[:, None]


def kernel(xyz1, xyz2, points1, points2, W1, b1, g1, be1, W2, b2, g2, be2):
    B, _, N = xyz1.shape
    D2 = points2.shape[1]
    bn = BN_BLK
    nj = N // bn

    idxT, wT = pl.pallas_call(
        _p1_kernel,
        grid=(B, nj),
        in_specs=[
            pl.BlockSpec((1, 3, bn), lambda b, j: (b, 0, j)),
            pl.BlockSpec((1, 3, S), lambda b, j: (b, 0, 0)),
        ],
        out_specs=[
            pl.BlockSpec((1, 3, bn), lambda b, j: (b, 0, j)),
            pl.BlockSpec((1, 3, bn), lambda b, j: (b, 0, j)),
        ],
        out_shape=[
            jax.ShapeDtypeStruct((B, 3, N), jnp.int32),
            jax.ShapeDtypeStruct((B, 3, N), jnp.float32),
        ],
    )(xyz1, xyz2)

    interp = _sc_interp(points2, idxT, wT, B, D2, N)

    h1, st1 = pl.pallas_call(
        functools.partial(_p2_kernel, nj=nj),
        grid=(B, nj),
        in_specs=[
            pl.BlockSpec((1, 128, bn), lambda b, j: (b, 0, j)),
            pl.BlockSpec((1, D2, bn), lambda b, j: (b, 0, j)),
            pl.BlockSpec((256, 384), lambda b, j: (0, 0)),
            pl.BlockSpec((256, 1), lambda b, j: (0, 0)),
        ],
        out_specs=[
            pl.BlockSpec((1, 256, bn), lambda b, j: (b, 0, j)),
            pl.BlockSpec((2, 256), lambda b, j: (0, 0)),
        ],
        out_shape=[
            jax.ShapeDtypeStruct((B, 256, N), jnp.float32),
            jax.ShapeDtypeStruct((2, 256), jnp.float32),
        ],
        scratch_shapes=[pltpu.VMEM((2, 256, BN_BLK), jnp.float32)],
    )(points1, interp, W1, b1[:, None])

    sc1, sh1 = _affine(st1, g1, be1, float(B * N))

    h2, st2 = pl.pallas_call(
        functools.partial(_p3_kernel, nj=nj),
        grid=(B, nj),
        in_specs=[
            pl.BlockSpec((1, 256, bn), lambda b, j: (b, 0, j)),
            pl.BlockSpec((128, 256), lambda b, j: (0, 0)),
            pl.BlockSpec((128, 1), lambda b, j: (0, 0)),
            pl.BlockSpec((256, 1), lambda b, j: (0, 0)),
            pl.BlockSpec((256, 1), lambda b, j: (0, 0)),
        ],
        out_specs=[
            pl.BlockSpec((1, 128, bn), lambda b, j: (b, 0, j)),
            pl.BlockSpec((2, 128), lambda b, j: (0, 0)),
        ],
        out_shape=[
            jax.ShapeDtypeStruct((B, 128, N), jnp.float32),
            jax.ShapeDtypeStruct((2, 128), jnp.float32),
        ],
        scratch_shapes=[pltpu.VMEM((2, 128, BN_BLK), jnp.float32)],
    )(h1, W2, b2[:, None], sc1, sh1)

    sc2, sh2 = _affine(st2, g2, be2, float(B * N))

    out = pl.pallas_call(
        _p4_kernel,
        grid=(B, nj),
        in_specs=[
            pl.BlockSpec((1, 128, bn), lambda b, j: (b, 0, j)),
            pl.BlockSpec((128, 1), lambda b, j: (0, 0)),
            pl.BlockSpec((128, 1), lambda b, j: (0, 0)),
        ],
        out_specs=pl.BlockSpec((1, 128, bn), lambda b, j: (b, 0, j)),
        out_shape=jax.ShapeDtypeStruct((B, 128, N), jnp.float32),
    )(h2, sc2, sh2)

    return out


# speedup vs baseline: 1.1375x; 1.1375x over previous
import functools
import jax
import jax.numpy as jnp
from jax import lax
from jax.experimental import pallas as pl
from jax.experimental.pallas import tpu as pltpu
from jax.experimental.pallas import tpu_sc as plsc

BN_BLK = 512
S = 1024
SC_CHUNK = 2048
HI = 3.4e38


def _top3(dists, iota_s):
    d = dists
    vals, idxs = [], []
    for k in range(3):
        m = jnp.min(d, axis=0)
        is_min = d == m[None, :]
        ik = jnp.min(jnp.where(is_min, iota_s, S), axis=0)
        vals.append(m)
        idxs.append(ik)
        if k < 2:
            d = jnp.where(iota_s == ik[None, :], HI, d)
    return vals, idxs


def _p1_kernel(x1_ref, x2_ref, idx_ref, w_ref):
    x1 = x1_ref[0]
    x2 = x2_ref[0]
    dots = jax.lax.dot_general(x2, x1, (((0,), (0,)), ((), ())),
                               preferred_element_type=jnp.float32,
                               precision=lax.Precision.DEFAULT)
    x1sq = (x1[0] * x1[0] + x1[1] * x1[1]) + x1[2] * x1[2]
    x2sq = (x2[0] * x2[0] + x2[1] * x2[1]) + x2[2] * x2[2]
    dists = (-2.0 * dots + x1sq[None, :]) + x2sq[:, None]

    iota_s = lax.broadcasted_iota(jnp.int32, dists.shape, 0)
    vals, idxs = _top3(dists, iota_s)

    r0 = 1.0 / (vals[0] + 1e-8)
    r1 = 1.0 / (vals[1] + 1e-8)
    r2 = 1.0 / (vals[2] + 1e-8)
    norm = r0 + r1 + r2

    idx_ref[0, 0] = idxs[0]
    idx_ref[0, 1] = idxs[1]
    idx_ref[0, 2] = idxs[2]
    w_ref[0, 0] = r0 / norm
    w_ref[0, 1] = r1 / norm
    w_ref[0, 2] = r2 / norm


def _sc_interp(p2, idxT, wT, B, D2, N):
    mesh = plsc.VectorSubcoreMesh(core_axis_name="c", subcore_axis_name="s")

    @functools.partial(
        pl.kernel,
        out_type=jax.ShapeDtypeStruct((B, D2, N), jnp.float32),
        mesh=mesh,
        compiler_params=pltpu.CompilerParams(use_tc_tiling_on_sc=False,
                                             needs_layout_passes=False),
        scratch_types=[
            pltpu.VMEM((16, S), jnp.float32),
            pltpu.VMEM((3, SC_CHUNK), jnp.int32),
            pltpu.VMEM((3, SC_CHUNK), jnp.float32),
            pltpu.VMEM((16, SC_CHUNK), jnp.float32),
        ],
    )
    def k(p2_hbm, idx_hbm, w_hbm, out_hbm, tbl_v, idx_v, w_v, o_v):
        wid = lax.axis_index("s") * 2 + lax.axis_index("c")
        b = wid % 2
        c = wid // 2
        pltpu.sync_copy(p2_hbm.at[b, pl.ds(c * 16, 16), :], tbl_v)
        fulls = [jax.lax.broadcast(jnp.int32(ch), (16,)) for ch in range(16)]

        def do_group(g):
            sl = pl.ds(g * 16, 16)
            i0 = idx_v[0, sl]
            i1 = idx_v[1, sl]
            i2 = idx_v[2, sl]
            w0 = w_v[0, sl]
            w1 = w_v[1, sl]
            w2 = w_v[2, sl]
            for ch in range(16):
                g0 = plsc.load_gather(tbl_v, [fulls[ch], i0])
                g1 = plsc.load_gather(tbl_v, [fulls[ch], i1])
                g2 = plsc.load_gather(tbl_v, [fulls[ch], i2])
                o_v[ch, sl] = (g0 * w0 + g1 * w1) + g2 * w2

        for t in range(N // SC_CHUNK):
            tsl = pl.ds(t * SC_CHUNK, SC_CHUNK)
            pltpu.sync_copy(idx_hbm.at[b, :, tsl], idx_v)
            pltpu.sync_copy(w_hbm.at[b, :, tsl], w_v)
            plsc.parallel_loop(0, SC_CHUNK // 16, 1, unroll=2)(do_group)
            pltpu.sync_copy(o_v, out_hbm.at[b, pl.ds(c * 16, 16), tsl])

    return k(p2, idxT, wT)


def _p2_kernel(p1_ref, it_ref, w1_ref, b1_ref, h1_ref, st1_ref, acc_ref, *, nj):
    b = pl.program_id(0)
    j = pl.program_id(1)
    W1 = w1_ref[...]
    h1 = (jax.lax.dot_general(W1[:, :128], p1_ref[0], (((1,), (0,)), ((), ())),
                              preferred_element_type=jnp.float32,
                              precision=lax.Precision.DEFAULT)
          + jax.lax.dot_general(W1[:, 128:], it_ref[0], (((1,), (0,)), ((), ())),
                                preferred_element_type=jnp.float32,
                                precision=lax.Precision.DEFAULT)
          + b1_ref[...])
    h1_ref[0] = h1

    @pl.when(jnp.logical_and(b == 0, j == 0))
    def _init():
        acc_ref[...] = jnp.zeros_like(acc_ref)

    acc_ref[0] += h1
    acc_ref[1] += h1 * h1

    @pl.when(jnp.logical_and(b == pl.num_programs(0) - 1, j == nj - 1))
    def _fin():
        st1_ref[0, :] = jnp.sum(acc_ref[0], axis=1)
        st1_ref[1, :] = jnp.sum(acc_ref[1], axis=1)


def _p3_kernel(h1_ref, w2_ref, b2_ref, sc1_ref, sh1_ref, h2_ref, st2_ref,
               acc_ref, *, nj):
    b = pl.program_id(0)
    j = pl.program_id(1)
    a1 = jnp.maximum(sc1_ref[...] * h1_ref[0] + sh1_ref[...], 0.0)
    h2 = (jax.lax.dot_general(w2_ref[...], a1, (((1,), (0,)), ((), ())),
                              preferred_element_type=jnp.float32,
                              precision=lax.Precision.DEFAULT)
          + b2_ref[...])
    h2_ref[0] = h2

    @pl.when(jnp.logical_and(b == 0, j == 0))
    def _init():
        acc_ref[...] = jnp.zeros_like(acc_ref)

    acc_ref[0] += h2
    acc_ref[1] += h2 * h2

    @pl.when(jnp.logical_and(b == pl.num_programs(0) - 1, j == nj - 1))
    def _fin():
        st2_ref[0, :] = jnp.sum(acc_ref[0], axis=1)
        st2_ref[1, :] = jnp.sum(acc_ref[1], axis=1)


def _p4_kernel(h2_ref, sc2_ref, sh2_ref, o_ref):
    o_ref[0] = jnp.maximum(sc2_ref[...] * h2_ref[0] + sh2_ref[...], 0.0)


def _affine(stats, g, be, count):
    mean = stats[0] / count
    var = stats[1] / count - mean * mean
    scale = g / jnp.sqrt(var + 1e-5)
    shift = be - scale * mean
    return scale[:, None], shift[:, None]


def kernel(xyz1, xyz2, points1, points2, W1, b1, g1, be1, W2, b2, g2, be2):
    B, _, N = xyz1.shape
    D2 = points2.shape[1]
    bn = BN_BLK
    nj = N // bn

    idxT, wT = pl.pallas_call(
        _p1_kernel,
        grid=(B, nj),
        in_specs=[
            pl.BlockSpec((1, 3, bn), lambda b, j: (b, 0, j)),
            pl.BlockSpec((1, 3, S), lambda b, j: (b, 0, 0)),
        ],
        out_specs=[
            pl.BlockSpec((1, 3, bn), lambda b, j: (b, 0, j)),
            pl.BlockSpec((1, 3, bn), lambda b, j: (b, 0, j)),
        ],
        out_shape=[
            jax.ShapeDtypeStruct((B, 3, N), jnp.int32),
            jax.ShapeDtypeStruct((B, 3, N), jnp.float32),
        ],
    )(xyz1, xyz2)

    interp = _sc_interp(points2, idxT, wT, B, D2, N)

    h1, st1 = pl.pallas_call(
        functools.partial(_p2_kernel, nj=nj),
        grid=(B, nj),
        in_specs=[
            pl.BlockSpec((1, 128, bn), lambda b, j: (b, 0, j)),
            pl.BlockSpec((1, D2, bn), lambda b, j: (b, 0, j)),
            pl.BlockSpec((256, 384), lambda b, j: (0, 0)),
            pl.BlockSpec((256, 1), lambda b, j: (0, 0)),
        ],
        out_specs=[
            pl.BlockSpec((1, 256, bn), lambda b, j: (b, 0, j)),
            pl.BlockSpec((2, 256), lambda b, j: (0, 0)),
        ],
        out_shape=[
            jax.ShapeDtypeStruct((B, 256, N), jnp.float32),
            jax.ShapeDtypeStruct((2, 256), jnp.float32),
        ],
        scratch_shapes=[pltpu.VMEM((2, 256, BN_BLK), jnp.float32)],
    )(points1, interp, W1, b1[:, None])

    sc1, sh1 = _affine(st1, g1, be1, float(B * N))

    h2, st2 = pl.pallas_call(
        functools.partial(_p3_kernel, nj=nj),
        grid=(B, nj),
        in_specs=[
            pl.BlockSpec((1, 256, bn), lambda b, j: (b, 0, j)),
            pl.BlockSpec((128, 256), lambda b, j: (0, 0)),
            pl.BlockSpec((128, 1), lambda b, j: (0, 0)),
            pl.BlockSpec((256, 1), lambda b, j: (0, 0)),
            pl.BlockSpec((256, 1), lambda b, j: (0, 0)),
        ],
        out_specs=[
            pl.BlockSpec((1, 128, bn), lambda b, j: (b, 0, j)),
            pl.BlockSpec((2, 128), lambda b, j: (0, 0)),
        ],
        out_shape=[
            jax.ShapeDtypeStruct((B, 128, N), jnp.float32),
            jax.ShapeDtypeStruct((2, 128), jnp.float32),
        ],
        scratch_shapes=[pltpu.VMEM((2, 128, BN_BLK), jnp.float32)],
    )(h1, W2, b2[:, None], sc1, sh1)

    sc2, sh2 = _affine(st2, g2, be2, float(B * N))

    out = pl.pallas_call(
        _p4_kernel,
        grid=(B, nj),
        in_specs=[
            pl.BlockSpec((1, 128, bn), lambda b, j: (b, 0, j)),
            pl.BlockSpec((128, 1), lambda b, j: (0, 0)),
            pl.BlockSpec((128, 1), lambda b, j: (0, 0)),
        ],
        out_specs=pl.BlockSpec((1, 128, bn), lambda b, j: (b, 0, j)),
        out_shape=jax.ShapeDtypeStruct((B, 128, N), jnp.float32),
    )(h2, sc2, sh2)

    return out
